# baseline (speedup 1.0000x reference)
import jax, jax.numpy as jnp
from jax.experimental import pallas as pl

def _head(h, Wp1, bp1, Wp2, bp2):
    n = h.shape[0]
    def body(h_ref, w1_ref, b1_ref, w2_ref, b2_ref, o_ref):
        z = jnp.dot(h_ref[...], w1_ref[...], preferred_element_type=jnp.float32) + b1_ref[...]
        z = jnp.dot(z, w2_ref[...], preferred_element_type=jnp.float32) + b2_ref[...]
        m = jnp.max(z, axis=1, keepdims=True)
        zs = z - m
        o_ref[...] = zs - jnp.log(jnp.sum(jnp.exp(zs), axis=1, keepdims=True))
    return pl.pallas_call(
        body, grid=(n // 400,),
        in_specs=[pl.BlockSpec((400, 32), lambda i: (i, 0)),
                  pl.BlockSpec((32, 32), lambda i: (0, 0)),
                  pl.BlockSpec((1, 32), lambda i: (0, 0)),
                  pl.BlockSpec((32, 7), lambda i: (0, 0)),
                  pl.BlockSpec((1, 7), lambda i: (0, 0))],
        out_specs=pl.BlockSpec((400, 7), lambda i: (i, 0)),
        out_shape=jax.ShapeDtypeStruct((n, 7), jnp.float32),
    )(h, Wp1, bp1.reshape(1, 32), Wp2, bp2.reshape(1, 7))

def _conv(x, src, dst, Wl, bl, Wr, n):
    h = x @ Wl
    msgs = jnp.take(h, src, axis=0)
    agg = jax.ops.segment_sum(msgs, dst, num_segments=n)
    deg = jax.ops.segment_sum(jnp.ones((src.shape[0],), h.dtype), dst, num_segments=n)
    return agg / jnp.clip(deg, 1.0, None)[:, None] + bl + x @ Wr

def kernel(x, edge_index, Wl1, bl1, Wr1, Wl2, bl2, Wr2, Wl3, bl3, Wr3, Wp1, bp1, Wp2, bp2):
    n = x.shape[0]
    src, dst = edge_index[0], edge_index[1]
    h = jax.nn.relu(_conv(x, src, dst, Wl1, bl1, Wr1, n))
    h = jax.nn.relu(_conv(h, src, dst, Wl2, bl2, Wr2, n))
    h = jax.nn.relu(_conv(h, src, dst, Wl3, bl3, Wr3, n))
    return _head(h, Wp1, bp1, Wp2, bp2)


# speedup vs baseline: 1.3418x; 1.3418x over previous
import functools

import jax
import jax.numpy as jnp
from jax import lax
from jax.experimental import pallas as pl
from jax.experimental.pallas import tpu as pltpu
from jax.experimental.pallas import tpu_sc as plsc

_NC = 2
_NS = 16
_NW = _NC * _NS
_CH = 128
_ROWB = 400


def _round_up(a, b):
    return (a + b - 1) // b * b


def _sc_gather_rows(table, src1, chunks):
    per_sub = chunks * _CH
    e_pad = per_sub * _NW
    mesh = plsc.VectorSubcoreMesh(core_axis_name="c", subcore_axis_name="s")

    @functools.partial(
        pl.kernel,
        out_type=jax.ShapeDtypeStruct((e_pad, 128), jnp.float32),
        mesh=mesh,
        scratch_types=[
            pltpu.VMEM((per_sub,), jnp.int32),
            pltpu.VMEM((_CH, 128), jnp.float32),
            pltpu.VMEM((_CH, 128), jnp.float32),
            pltpu.SemaphoreType.DMA,
            pltpu.SemaphoreType.DMA,
            pltpu.SemaphoreType.DMA,
        ],
    )
    def k(h_hbm, src_hbm, out_hbm, idx_v, rows_a, rows_b, isem, gsa, gsb):
        c = lax.axis_index("c")
        s = lax.axis_index("s")
        wid = s * _NC + c
        base_e = wid * per_sub

        pltpu.async_copy(src_hbm.at[pl.ds(base_e, per_sub)], idx_v,
                         isem).wait()

        pltpu.async_copy(h_hbm.at[idx_v.at[pl.ds(0, _CH)]], rows_a, gsa)

        @pl.loop(0, chunks, step=2)
        def _(j):
            @pl.when(j + 1 < chunks)
            def _():
                pltpu.async_copy(
                    h_hbm.at[idx_v.at[pl.ds((j + 1) * _CH, _CH)]], rows_b,
                    gsb)

            pltpu.make_async_copy(h_hbm.at[idx_v.at[pl.ds(j * _CH, _CH)]],
                                  rows_a, gsa).wait()
            pltpu.async_copy(
                rows_a, out_hbm.at[pl.ds(base_e + j * _CH, _CH)], gsa).wait()

            @pl.when(j + 2 < chunks)
            def _():
                pltpu.async_copy(
                    h_hbm.at[idx_v.at[pl.ds((j + 2) * _CH, _CH)]], rows_a,
                    gsa)

            pltpu.make_async_copy(
                h_hbm.at[idx_v.at[pl.ds((j + 1) * _CH, _CH)]], rows_b,
                gsb).wait()
            pltpu.async_copy(
                rows_b, out_hbm.at[pl.ds(base_e + (j + 1) * _CH, _CH)],
                gsb).wait()

    return k(table, src1)


def _tc_proj1(x, w1):
    n, d = x.shape

    def body(x_ref, w_ref, h_ref):
        h_ref[...] = jnp.dot(x_ref[...], w_ref[...],
                             preferred_element_type=jnp.float32)

    return pl.pallas_call(
        body,
        grid=(n // _ROWB,),
        in_specs=[
            pl.BlockSpec((_ROWB, d), lambda i: (i, 0)),
            pl.BlockSpec((d, 128), lambda i: (0, 0)),
        ],
        out_specs=pl.BlockSpec((_ROWB, 128), lambda i: (i, 0)),
        out_shape=jax.ShapeDtypeStruct((n, 128), jnp.float32),
    )(x, w1)


def _tc_combine1(p0, p1, degp, bl1, h1, w2):
    n = h1.shape[0]

    def body(p0_ref, p1_ref, dg_ref, bl_ref, h1_ref, w_ref, h2_ref, inv_ref):
        agg = jnp.concatenate(
            [p0_ref[0] + p0_ref[1], p1_ref[0] + p1_ref[1]], axis=1)
        deg = dg_ref[0, :, 0] + dg_ref[1, :, 0]
        inv = 1.0 / jnp.maximum(deg, 1.0)
        x2 = jnp.maximum(agg * inv[:, None] + bl_ref[...] + h1_ref[:, 64:],
                         0.0)
        h2 = jnp.dot(x2, w_ref[...], preferred_element_type=jnp.float32)
        h2_ref[...] = jnp.concatenate(
            [h2, jnp.zeros((h2.shape[0], 64), jnp.float32)], axis=1)
        inv_ref[...] = inv[:, None]

    return pl.pallas_call(
        body,
        grid=(n // _ROWB,),
        in_specs=[
            pl.BlockSpec((2, _ROWB, 32), lambda i: (0, i, 0)),
            pl.BlockSpec((2, _ROWB, 32), lambda i: (0, i, 0)),
            pl.BlockSpec((2, _ROWB, 16), lambda i: (0, i, 0)),
            pl.BlockSpec((1, 64), lambda i: (0, 0)),
            pl.BlockSpec((_ROWB, 128), lambda i: (i, 0)),
            pl.BlockSpec((64, 64), lambda i: (0, 0)),
        ],
        out_specs=[
            pl.BlockSpec((_ROWB, 128), lambda i: (i, 0)),
            pl.BlockSpec((_ROWB, 1), lambda i: (i, 0)),
        ],
        out_shape=[
            jax.ShapeDtypeStruct((n, 128), jnp.float32),
            jax.ShapeDtypeStruct((n, 1), jnp.float32),
        ],
    )(p0, p1, degp, bl1, h1, w2)


def _tc_combine2(p, inv, bl, h_prev, w_next):
    n = h_prev.shape[0]

    def body(p_ref, inv_ref, bl_ref, hp_ref, w_ref, h_ref):
        agg = p_ref[0] + p_ref[1]
        x_ = jnp.maximum(agg * inv_ref[...] + bl_ref[...] + hp_ref[:, 32:64],
                         0.0)
        h = jnp.dot(x_, w_ref[...], preferred_element_type=jnp.float32)
        h_ref[...] = jnp.concatenate(
            [h, jnp.zeros((h.shape[0], 64), jnp.float32)], axis=1)

    return pl.pallas_call(
        body,
        grid=(n // _ROWB,),
        in_specs=[
            pl.BlockSpec((2, _ROWB, 32), lambda i: (0, i, 0)),
            pl.BlockSpec((_ROWB, 1), lambda i: (i, 0)),
            pl.BlockSpec((1, 32), lambda i: (0, 0)),
            pl.BlockSpec((_ROWB, 128), lambda i: (i, 0)),
            pl.BlockSpec((32, 64), lambda i: (0, 0)),
        ],
        out_specs=pl.BlockSpec((_ROWB, 128), lambda i: (i, 0)),
        out_shape=jax.ShapeDtypeStruct((n, 128), jnp.float32),
    )(p, inv, bl, h_prev, w_next)


def _tc_head(p, inv, bl, h_prev, wp1, bp1, wp2, bp2):
    n = h_prev.shape[0]

    def body(p_ref, inv_ref, bl_ref, hp_ref, wp1_ref, bp1_ref, wp2_ref,
             bp2_ref, out_ref):
        agg = p_ref[0] + p_ref[1]
        x_ = jnp.maximum(agg * inv_ref[...] + bl_ref[...] + hp_ref[:, 32:64],
                         0.0)
        h = jnp.dot(x_, wp1_ref[...],
                    preferred_element_type=jnp.float32) + bp1_ref[...]
        z = jnp.dot(h, wp2_ref[...],
                    preferred_element_type=jnp.float32) + bp2_ref[...]
        m = jnp.max(z, axis=1, keepdims=True)
        zs = z - m
        out_ref[...] = zs - jnp.log(jnp.sum(jnp.exp(zs), axis=1,
                                            keepdims=True))

    return pl.pallas_call(
        body,
        grid=(n // _ROWB,),
        in_specs=[
            pl.BlockSpec((2, _ROWB, 32), lambda i: (0, i, 0)),
            pl.BlockSpec((_ROWB, 1), lambda i: (i, 0)),
            pl.BlockSpec((1, 32), lambda i: (0, 0)),
            pl.BlockSpec((_ROWB, 128), lambda i: (i, 0)),
            pl.BlockSpec((32, 32), lambda i: (0, 0)),
            pl.BlockSpec((1, 32), lambda i: (0, 0)),
            pl.BlockSpec((32, 7), lambda i: (0, 0)),
            pl.BlockSpec((1, 7), lambda i: (0, 0)),
        ],
        out_specs=pl.BlockSpec((_ROWB, 7), lambda i: (i, 0)),
        out_shape=jax.ShapeDtypeStruct((n, 7), jnp.float32),
    )(p, inv, bl, h_prev, wp1, bp1, wp2, bp2)


def kernel(x, edge_index, Wl1, bl1, Wr1, Wl2, bl2, Wr2, Wl3, bl3, Wr3,
           Wp1, bp1, Wp2, bp2):
    n = x.shape[0]
    e = edge_index.shape[1]
    n_pad = _NS * _round_up(-(-n // _NS), 8)
    e_pad = _round_up(e, _NW * _CH * 2)
    chunks = e_pad // (_NW * _CH)
    src = edge_index[0]
    dst = edge_index[1]
    src1 = jnp.concatenate([src, jnp.zeros((e_pad - e,), jnp.int32)])

    w1 = jnp.concatenate([Wl1, Wr1], axis=1)
    w2 = jnp.concatenate([Wl2, Wr2], axis=1)
    w3 = jnp.concatenate([Wl3, Wr3], axis=1)

    degv = jax.ops.segment_sum(jnp.ones((e,), jnp.float32), dst,
                               num_segments=n)
    degp = jnp.zeros((2, n_pad, 16), jnp.float32).at[0, :n, 0].set(degv)

    def agg_pad(msgs, width):
        a = jax.ops.segment_sum(msgs[:e, :width], dst, num_segments=n)
        return jnp.zeros((2, n_pad, 32), jnp.float32).at[0, :n, :].set(
            a if width == 32 else a[:, :32]), a

    h1 = _tc_proj1(x, w1)
    msgs1 = _sc_gather_rows(h1, src1, chunks)
    a1 = jax.ops.segment_sum(msgs1[:e, :64], dst, num_segments=n)
    zp = jnp.zeros((2, n_pad, 32), jnp.float32)
    p0 = zp.at[0, :n, :].set(a1[:, :32])
    p1 = zp.at[0, :n, :].set(a1[:, 32:])
    h2, inv = _tc_combine1(p0, p1, degp, bl1.reshape(1, 64), h1, w2)

    msgs2 = _sc_gather_rows(h2, src1, chunks)
    a2 = jax.ops.segment_sum(msgs2[:e, :32], dst, num_segments=n)
    p2 = zp.at[0, :n, :].set(a2)
    h3 = _tc_combine2(p2, inv, bl2.reshape(1, 32), h2, w3)

    msgs3 = _sc_gather_rows(h3, src1, chunks)
    a3 = jax.ops.segment_sum(msgs3[:e, :32], dst, num_segments=n)
    p3 = zp.at[0, :n, :].set(a3)
    return _tc_head(p3, inv, bl3.reshape(1, 32), h3, Wp1,
                    bp1.reshape(1, 32), Wp2, bp2.reshape(1, 7))


# speedup vs baseline: 1.4794x; 1.1026x over previous
import functools

import jax
import jax.numpy as jnp
from jax import lax
from jax.experimental import pallas as pl
from jax.experimental.pallas import tpu as pltpu
from jax.experimental.pallas import tpu_sc as plsc

_NC = 2
_NS = 16
_NW = _NC * _NS
_CH = 128
_ROWB = 400


def _round_up(a, b):
    return (a + b - 1) // b * b


def _sc_gather_rows(table, src1, chunks):
    per_sub = chunks * _CH
    e_pad = per_sub * _NW
    mesh = plsc.VectorSubcoreMesh(core_axis_name="c", subcore_axis_name="s")

    @functools.partial(
        pl.kernel,
        out_type=jax.ShapeDtypeStruct((e_pad, 128), jnp.float32),
        mesh=mesh,
        scratch_types=[
            pltpu.VMEM((per_sub,), jnp.int32),
            pltpu.VMEM((_CH, 128), jnp.float32),
            pltpu.VMEM((_CH, 128), jnp.float32),
            pltpu.SemaphoreType.DMA,
            pltpu.SemaphoreType.DMA,
            pltpu.SemaphoreType.DMA,
        ],
    )
    def k(h_hbm, src_hbm, out_hbm, idx_v, rows_a, rows_b, isem, gsa, gsb):
        c = lax.axis_index("c")
        s = lax.axis_index("s")
        wid = s * _NC + c
        base_e = wid * per_sub

        pltpu.async_copy(src_hbm.at[pl.ds(base_e, per_sub)], idx_v,
                         isem).wait()

        pltpu.async_copy(h_hbm.at[idx_v.at[pl.ds(0, _CH)]], rows_a, gsa)

        @pl.loop(0, chunks, step=2)
        def _(j):
            @pl.when(j + 1 < chunks)
            def _():
                pltpu.async_copy(
                    h_hbm.at[idx_v.at[pl.ds((j + 1) * _CH, _CH)]], rows_b,
                    gsb)

            pltpu.make_async_copy(h_hbm.at[idx_v.at[pl.ds(j * _CH, _CH)]],
                                  rows_a, gsa).wait()
            pltpu.async_copy(
                rows_a, out_hbm.at[pl.ds(base_e + j * _CH, _CH)], gsa).wait()

            @pl.when(j + 2 < chunks)
            def _():
                pltpu.async_copy(
                    h_hbm.at[idx_v.at[pl.ds((j + 2) * _CH, _CH)]], rows_a,
                    gsa)

            pltpu.make_async_copy(
                h_hbm.at[idx_v.at[pl.ds((j + 1) * _CH, _CH)]], rows_b,
                gsb).wait()
            pltpu.async_copy(
                rows_b, out_hbm.at[pl.ds(base_e + (j + 1) * _CH, _CH)],
                gsb).wait()

    return k(table, src1)


def _tc_proj1(x, w1):
    n, d = x.shape

    def body(x_ref, w_ref, h_ref):
        h_ref[...] = jnp.dot(x_ref[...], w_ref[...],
                             preferred_element_type=jnp.float32)

    return pl.pallas_call(
        body,
        grid=(n // _ROWB,),
        in_specs=[
            pl.BlockSpec((_ROWB, d), lambda i: (i, 0)),
            pl.BlockSpec((d, 128), lambda i: (0, 0)),
        ],
        out_specs=pl.BlockSpec((_ROWB, 128), lambda i: (i, 0)),
        out_shape=jax.ShapeDtypeStruct((n, 128), jnp.float32),
    )(x, w1)


def _tc_combine1(p0, p1, degp, bl1, h1, w2):
    n = h1.shape[0]

    def body(p0_ref, p1_ref, dg_ref, bl_ref, h1_ref, w_ref, h2_ref, inv_ref):
        agg = jnp.concatenate(
            [p0_ref[0] + p0_ref[1], p1_ref[0] + p1_ref[1]], axis=1)
        deg = dg_ref[0, :, 0] + dg_ref[1, :, 0]
        inv = 1.0 / jnp.maximum(deg, 1.0)
        x2 = jnp.maximum(agg * inv[:, None] + bl_ref[...] + h1_ref[:, 64:],
                         0.0)
        h2 = jnp.dot(x2, w_ref[...], preferred_element_type=jnp.float32)
        h2_ref[...] = jnp.concatenate(
            [h2, jnp.zeros((h2.shape[0], 64), jnp.float32)], axis=1)
        inv_ref[...] = inv[:, None]

    return pl.pallas_call(
        body,
        grid=(n // _ROWB,),
        in_specs=[
            pl.BlockSpec((2, _ROWB, 32), lambda i: (0, i, 0)),
            pl.BlockSpec((2, _ROWB, 32), lambda i: (0, i, 0)),
            pl.BlockSpec((2, _ROWB, 16), lambda i: (0, i, 0)),
            pl.BlockSpec((1, 64), lambda i: (0, 0)),
            pl.BlockSpec((_ROWB, 128), lambda i: (i, 0)),
            pl.BlockSpec((64, 64), lambda i: (0, 0)),
        ],
        out_specs=[
            pl.BlockSpec((_ROWB, 128), lambda i: (i, 0)),
            pl.BlockSpec((_ROWB, 1), lambda i: (i, 0)),
        ],
        out_shape=[
            jax.ShapeDtypeStruct((n, 128), jnp.float32),
            jax.ShapeDtypeStruct((n, 1), jnp.float32),
        ],
    )(p0, p1, degp, bl1, h1, w2)


def _tc_combine2(p, inv, bl, h_prev, w_next):
    n = h_prev.shape[0]

    def body(p_ref, inv_ref, bl_ref, hp_ref, w_ref, h_ref):
        agg = p_ref[0] + p_ref[1]
        x_ = jnp.maximum(agg * inv_ref[...] + bl_ref[...] + hp_ref[:, 32:64],
                         0.0)
        h = jnp.dot(x_, w_ref[...], preferred_element_type=jnp.float32)
        h_ref[...] = jnp.concatenate(
            [h, jnp.zeros((h.shape[0], 64), jnp.float32)], axis=1)

    return pl.pallas_call(
        body,
        grid=(n // _ROWB,),
        in_specs=[
            pl.BlockSpec((2, _ROWB, 32), lambda i: (0, i, 0)),
            pl.BlockSpec((_ROWB, 1), lambda i: (i, 0)),
            pl.BlockSpec((1, 32), lambda i: (0, 0)),
            pl.BlockSpec((_ROWB, 128), lambda i: (i, 0)),
            pl.BlockSpec((32, 64), lambda i: (0, 0)),
        ],
        out_specs=pl.BlockSpec((_ROWB, 128), lambda i: (i, 0)),
        out_shape=jax.ShapeDtypeStruct((n, 128), jnp.float32),
    )(p, inv, bl, h_prev, w_next)


def _tc_head(p, inv, bl, h_prev, wp1, bp1, wp2, bp2):
    n = h_prev.shape[0]

    def body(p_ref, inv_ref, bl_ref, hp_ref, wp1_ref, bp1_ref, wp2_ref,
             bp2_ref, out_ref):
        agg = p_ref[0] + p_ref[1]
        x_ = jnp.maximum(agg * inv_ref[...] + bl_ref[...] + hp_ref[:, 32:64],
                         0.0)
        h = jnp.dot(x_, wp1_ref[...],
                    preferred_element_type=jnp.float32) + bp1_ref[...]
        z = jnp.dot(h, wp2_ref[...],
                    preferred_element_type=jnp.float32) + bp2_ref[...]
        m = jnp.max(z, axis=1, keepdims=True)
        zs = z - m
        out_ref[...] = zs - jnp.log(jnp.sum(jnp.exp(zs), axis=1,
                                            keepdims=True))

    return pl.pallas_call(
        body,
        grid=(n // _ROWB,),
        in_specs=[
            pl.BlockSpec((2, _ROWB, 32), lambda i: (0, i, 0)),
            pl.BlockSpec((_ROWB, 1), lambda i: (i, 0)),
            pl.BlockSpec((1, 32), lambda i: (0, 0)),
            pl.BlockSpec((_ROWB, 128), lambda i: (i, 0)),
            pl.BlockSpec((32, 32), lambda i: (0, 0)),
            pl.BlockSpec((1, 32), lambda i: (0, 0)),
            pl.BlockSpec((32, 7), lambda i: (0, 0)),
            pl.BlockSpec((1, 7), lambda i: (0, 0)),
        ],
        out_specs=pl.BlockSpec((_ROWB, 7), lambda i: (i, 0)),
        out_shape=jax.ShapeDtypeStruct((n, 7), jnp.float32),
    )(p, inv, bl, h_prev, wp1, bp1, wp2, bp2)


def kernel(x, edge_index, Wl1, bl1, Wr1, Wl2, bl2, Wr2, Wl3, bl3, Wr3,
           Wp1, bp1, Wp2, bp2):
    n = x.shape[0]
    e = edge_index.shape[1]
    n_pad = _NS * _round_up(-(-n // _NS), 8)
    e_pad = _round_up(e, _NW * _CH * 2)
    chunks = e_pad // (_NW * _CH)
    perm = jnp.argsort(edge_index[1])
    src = edge_index[0][perm]
    dst = edge_index[1][perm]
    src1 = jnp.concatenate([src, jnp.zeros((e_pad - e,), jnp.int32)])

    w1 = jnp.concatenate([Wl1, Wr1], axis=1)
    w2 = jnp.concatenate([Wl2, Wr2], axis=1)
    w3 = jnp.concatenate([Wl3, Wr3], axis=1)

    degv = jax.ops.segment_sum(jnp.ones((e,), jnp.float32), dst,
                               num_segments=n, indices_are_sorted=True)
    degp = jnp.zeros((2, n_pad, 16), jnp.float32).at[0, :n, 0].set(degv)

    h1 = _tc_proj1(x, w1)
    msgs1 = _sc_gather_rows(h1, src1, chunks)
    a1 = jax.ops.segment_sum(msgs1[:e, :64], dst, num_segments=n,
                             indices_are_sorted=True)
    zp = jnp.zeros((2, n_pad, 32), jnp.float32)
    p0 = zp.at[0, :n, :].set(a1[:, :32])
    p1 = zp.at[0, :n, :].set(a1[:, 32:])
    h2, inv = _tc_combine1(p0, p1, degp, bl1.reshape(1, 64), h1, w2)

    msgs2 = _sc_gather_rows(h2, src1, chunks)
    a2 = jax.ops.segment_sum(msgs2[:e, :32], dst, num_segments=n,
                             indices_are_sorted=True)
    p2 = zp.at[0, :n, :].set(a2)
    h3 = _tc_combine2(p2, inv, bl2.reshape(1, 32), h2, w3)

    msgs3 = _sc_gather_rows(h3, src1, chunks)
    a3 = jax.ops.segment_sum(msgs3[:e, :32], dst, num_segments=n,
                             indices_are_sorted=True)
    p3 = zp.at[0, :n, :].set(a3)
    return _tc_head(p3, inv, bl3.reshape(1, 32), h3, Wp1,
                    bp1.reshape(1, 32), Wp2, bp2.reshape(1, 7))
